# baseline (speedup 1.0000x reference)
import jax
import jax.numpy as jnp
from jax import lax
from jax.experimental import pallas as pl
from jax.experimental.pallas import tpu as pltpu
from jax.experimental.pallas import tpu_sc as plsc

_NC = 2
_NS = 16
_NW = _NC * _NS
_L = 16

_D = 64
_CHUNK = 128


def _rsqrt(x):
    i = plsc.bitcast(x, jnp.int32)
    i = jnp.int32(0x5F3759DF) - lax.shift_right_logical(i, 1)
    y = plsc.bitcast(i, jnp.float32)
    for _ in range(3):
        y = y * (1.5 - 0.5 * x * y * y)
    return y


def _body(users, pos_items, neg_items, relations, ent_emb, rel_emb,
          out_pos, out_neg,
          idx_u, idx_p, idx_n, idx_r,
          rows_u, rows_p, rows_n, rows_r,
          outp_v, outn_v, sem):
    wid = lax.axis_index("s") * _NC + lax.axis_index("c")
    per_w = out_pos.shape[0] // _NW
    n_chunks = per_w // _CHUNK

    for c in range(n_chunks):
        base = wid * per_w + c * _CHUNK
        pltpu.sync_copy(users.at[pl.ds(base, _CHUNK)], idx_u)
        pltpu.sync_copy(pos_items.at[pl.ds(base, _CHUNK)], idx_p)
        pltpu.sync_copy(neg_items.at[pl.ds(base, _CHUNK)], idx_n)
        pltpu.sync_copy(relations.at[pl.ds(base, _CHUNK)], idx_r)

        cp_u = pltpu.async_copy(ent_emb.at[idx_u], rows_u, sem)
        cp_p = pltpu.async_copy(ent_emb.at[idx_p], rows_p, sem)
        cp_n = pltpu.async_copy(ent_emb.at[idx_n], rows_n, sem)
        cp_r = pltpu.async_copy(rel_emb.at[idx_r], rows_r, sem)
        cp_u.wait()
        cp_p.wait()
        cp_n.wait()
        cp_r.wait()

        lane_iota = lax.iota(jnp.int32, _L)

        def group(g, _):
            resp = jnp.zeros((_L,), jnp.float32)
            resn = jnp.zeros((_L,), jnp.float32)
            for j in range(_L):
                e = g * _L + j
                accp = jnp.zeros((_L,), jnp.float32)
                accn = jnp.zeros((_L,), jnp.float32)
                for k in range(_D // _L):
                    sl = pl.ds(k * _L, _L)
                    u = rows_u[e, sl]
                    r = rows_r[e, sl]
                    p = rows_p[e, sl]
                    n = rows_n[e, sl]
                    t = u + r
                    dp = t - p
                    dn = t - n
                    accp = accp + dp * dp
                    accn = accn + dn * dn
                lane = lane_iota == j
                resp = jnp.where(lane, jnp.sum(accp), resp)
                resn = jnp.where(lane, jnp.sum(accn), resn)
            sl = pl.ds(g * _L, _L)
            outp_v[sl] = -(resp * _rsqrt(jnp.maximum(resp, 1e-30)))
            outn_v[sl] = -(resn * _rsqrt(jnp.maximum(resn, 1e-30)))
            return 0

        lax.fori_loop(0, _CHUNK // _L, group, 0)

        pltpu.sync_copy(outp_v, out_pos.at[pl.ds(base, _CHUNK)])
        pltpu.sync_copy(outn_v, out_neg.at[pl.ds(base, _CHUNK)])


def kernel(users, pos_items, neg_items, relations, ent_emb, rel_emb):
    B = users.shape[0]
    users = users.astype(jnp.int32)
    pos_items = pos_items.astype(jnp.int32)
    neg_items = neg_items.astype(jnp.int32)
    relations = relations.astype(jnp.int32)

    run = pl.kernel(
        _body,
        out_type=(
            jax.ShapeDtypeStruct((B,), jnp.float32),
            jax.ShapeDtypeStruct((B,), jnp.float32),
        ),
        mesh=plsc.VectorSubcoreMesh(
            core_axis_name="c", subcore_axis_name="s",
            num_cores=_NC, num_subcores=_NS,
        ),
        compiler_params=pltpu.CompilerParams(
            needs_layout_passes=False, use_tc_tiling_on_sc=False,
        ),
        scratch_types=[
            pltpu.VMEM((_CHUNK,), jnp.int32),
            pltpu.VMEM((_CHUNK,), jnp.int32),
            pltpu.VMEM((_CHUNK,), jnp.int32),
            pltpu.VMEM((_CHUNK,), jnp.int32),
            pltpu.VMEM((_CHUNK, _D), jnp.float32),
            pltpu.VMEM((_CHUNK, _D), jnp.float32),
            pltpu.VMEM((_CHUNK, _D), jnp.float32),
            pltpu.VMEM((_CHUNK, _D), jnp.float32),
            pltpu.VMEM((_CHUNK,), jnp.float32),
            pltpu.VMEM((_CHUNK,), jnp.float32),
            pltpu.SemaphoreType.DMA,
        ],
    )
    return run(users, pos_items, neg_items, relations, ent_emb, rel_emb)
